# Initial kernel scaffold; baseline (speedup 1.0000x reference)
import jax
import jax.numpy as jnp
from jax.experimental import pallas as pl


def kernel(x, edge_index, W1, b1, gW, gb, W2, b2, W3, b3, cW1, cb1, cW2, cb2, cW3, cb3):
    raise NotImplementedError("write your pallas kernel here")


# speedup vs baseline: 15.9337x; 15.9337x over previous
import functools

import jax
import jax.numpy as jnp
from jax import lax
from jax.experimental import pallas as pl
from jax.experimental.pallas import tpu as pltpu
from jax.experimental.pallas import tpu_sc as plsc

N_NODES = 10000
H = 32
QW = 8
NQ = H // QW
DW = 8

NC = 2
NS = 16
NW = NC * NS
CH = 128
CB = 8
NPAD = 10112
NSP = 10016
RPS = NPAD // NS
LAST = NS - 1
RLAST = NSP - RPS * LAST


def _mesh():
    return plsc.VectorSubcoreMesh(core_axis_name="c", subcore_axis_name="s",
                                  num_cores=NC, num_subcores=NS)


def _tile_slab(s, do):
    @pl.when(s < LAST)
    def _():
        do(s * RPS, RPS)

    @pl.when(s == LAST)
    def _():
        do(LAST * RPS, RLAST)


def _deg_sc(dstb, ones, zeros, kchunks):
    kb = kchunks // CB

    def body(dstb_hbm, ones_hbm, zeros_hbm, out_hbm,
             dst_v, ones_v, buf_v, acc_sh):
        c = lax.axis_index("c")
        s = lax.axis_index("s")
        wid = c * NS + s
        pltpu.sync_copy(ones_hbm, ones_v)
        pltpu.sync_copy(zeros_hbm, buf_v)

        def zacc(start, size):
            pltpu.sync_copy(buf_v.at[pl.ds(0, size)],
                            acc_sh.at[pl.ds(start, size)])

        _tile_slab(s, zacc)
        plsc.subcore_barrier()

        def block(j, carry):
            pltpu.sync_copy(dstb_hbm.at[wid, j], dst_v)
            for r in range(CB):
                pltpu.sync_copy(ones_v, acc_sh.at[dst_v.at[r]], add=True)
            return carry

        lax.fori_loop(0, kb, block, 0)
        plsc.subcore_barrier()

        def wout(start, size):
            pltpu.sync_copy(acc_sh.at[pl.ds(start, size)],
                            buf_v.at[pl.ds(0, size)])
            pltpu.sync_copy(buf_v.at[pl.ds(0, size)],
                            out_hbm.at[c, pl.ds(start, size)])

        _tile_slab(s, wout)

    fn = pl.kernel(
        body,
        out_type=jax.ShapeDtypeStruct((NC, NPAD, DW), jnp.float32),
        mesh=_mesh(),
        compiler_params=pltpu.CompilerParams(use_tc_tiling_on_sc=False),
        scratch_types=[
            pltpu.VMEM((CB, CH), jnp.int32),
            pltpu.VMEM((CH, DW), jnp.float32),
            pltpu.VMEM((RPS, DW), jnp.float32),
            pltpu.VMEM_SHARED((NSP, DW), jnp.float32),
        ],
    )
    return fn(dstb, ones, zeros)


@functools.lru_cache(maxsize=None)
def _agg_fn(kchunks):
    kb = kchunks // CB

    def body(table_hbm, srcb_hbm, dstb_hbm, zeros_hbm, out_hbm,
             src_v, dst_v, rows_v, buf_v, zero_v, tab_sh, acc_sh):
        c = lax.axis_index("c")
        s = lax.axis_index("s")
        wid = c * NS + s
        pltpu.sync_copy(zeros_hbm, zero_v)

        for p in range(NQ):
            def stage(start, size):
                pltpu.sync_copy(table_hbm.at[p, pl.ds(start, size)],
                                buf_v.at[pl.ds(0, size)])
                pltpu.sync_copy(buf_v.at[pl.ds(0, size)],
                                tab_sh.at[pl.ds(start, size)])

            def zacc(start, size):
                pltpu.sync_copy(zero_v.at[pl.ds(0, size)],
                                acc_sh.at[pl.ds(start, size)])

            _tile_slab(s, stage)
            _tile_slab(s, zacc)
            plsc.subcore_barrier()

            def block(j, carry):
                pltpu.sync_copy(srcb_hbm.at[wid, j], src_v)
                pltpu.sync_copy(dstb_hbm.at[wid, j], dst_v)
                for r in range(CB):
                    pltpu.sync_copy(tab_sh.at[src_v.at[r]], rows_v)
                    pltpu.sync_copy(rows_v, acc_sh.at[dst_v.at[r]],
                                    add=True)
                return carry

            lax.fori_loop(0, kb, block, 0)
            plsc.subcore_barrier()

            def wout(start, size):
                pltpu.sync_copy(acc_sh.at[pl.ds(start, size)],
                                buf_v.at[pl.ds(0, size)])
                pltpu.sync_copy(buf_v.at[pl.ds(0, size)],
                                out_hbm.at[p, c, pl.ds(start, size)])

            _tile_slab(s, wout)
            if p != NQ - 1:
                plsc.subcore_barrier()

    return pl.kernel(
        body,
        out_type=jax.ShapeDtypeStruct((NQ, NC, NPAD, QW), jnp.float32),
        mesh=_mesh(),
        compiler_params=pltpu.CompilerParams(use_tc_tiling_on_sc=False),
        scratch_types=[
            pltpu.VMEM((CB, CH), jnp.int32),
            pltpu.VMEM((CB, CH), jnp.int32),
            pltpu.VMEM((CH, QW), jnp.float32),
            pltpu.VMEM((RPS, QW), jnp.float32),
            pltpu.VMEM((RPS, QW), jnp.float32),
            pltpu.VMEM_SHARED((NSP, QW), jnp.float32),
            pltpu.VMEM_SHARED((NSP, QW), jnp.float32),
        ],
    )


def _agg_sc(tabs, srcb, dstb, zeros, kchunks):
    return _agg_fn(kchunks)(tabs, srcb, dstb, zeros)


def _ct(a, b):
    return lax.dot_general(a, b, (((1,), (1,)), ((), ())),
                           preferred_element_type=jnp.float32)


def _split(hs, out_ref):
    for p in range(NQ):
        out_ref[p] = hs[:, p * QW:(p + 1) * QW]


def _join_agg(aggp):
    return jnp.concatenate([aggp[p, 0] + aggp[p, 1] for p in range(NQ)],
                           axis=1)


def _join_tab(tabs):
    return jnp.concatenate([tabs[p] for p in range(NQ)], axis=1)


def _prep_body(x_ref, w1_ref, degp_ref, tab1_ref, dinv_ref):
    degq = degp_ref[0] + degp_ref[1]
    rep = jnp.full((H, DW), 1.0 / DW, jnp.float32)
    deg = _ct(degq, rep) + 1.0
    dinv = lax.rsqrt(deg)
    h = _ct(x_ref[...], w1_ref[...])
    _split(h * dinv, tab1_ref)
    dinv_ref[...] = dinv


def _mid1_body(aggp_ref, tab1_ref, dinv_ref, b1_ref, gw_ref, gb_ref, w2_ref,
               tab2_ref):
    dinv = dinv_ref[...]
    h1 = jax.nn.relu((_join_agg(aggp_ref[...]) + _join_tab(tab1_ref[...]))
                     * dinv + b1_ref[...])
    gate = jax.nn.sigmoid(_ct(h1, gw_ref[...]) + gb_ref[...])
    h1g = h1 * gate
    _split(_ct(h1g, w2_ref[...]) * dinv, tab2_ref)


def _mid2_body(aggp_ref, tab2_ref, dinv_ref, b2_ref, w3_ref, tab3_ref):
    dinv = dinv_ref[...]
    h2 = jax.nn.relu((_join_agg(aggp_ref[...]) + _join_tab(tab2_ref[...]))
                     * dinv + b2_ref[...])
    _split(_ct(h2, w3_ref[...]) * dinv, tab3_ref)


def _gsum_body(aggp_ref, tab3_ref, dinv_ref, b3_ref, gsum_ref):
    i = pl.program_id(0)
    h3 = ((_join_agg(aggp_ref[...]) + _join_tab(tab3_ref[...]))
          * dinv_ref[...] + b3_ref[...])
    rows = lax.broadcasted_iota(jnp.int32, (BR, H), 0) + i * BR
    h3 = jnp.where(rows < N_NODES, h3, 0.0)
    part = jnp.sum(h3, axis=0, keepdims=True) * (1.0 / N_NODES)

    @pl.when(i == 0)
    def _():
        gsum_ref[...] = part

    @pl.when(i > 0)
    def _():
        gsum_ref[...] = gsum_ref[...] + part


def _head_body(g_ref, cw1_ref, cb1_ref, cw2_ref, cb2_ref, cw3_ref, cb3_ref,
               out_ref):
    z = jax.nn.relu(_ct(g_ref[...], cw1_ref[...]) + cb1_ref[...])
    z = jax.nn.relu(_ct(z, cw2_ref[...]) + cb2_ref[...])
    z = _ct(z, cw3_ref[...]) + cb3_ref[...]
    e = jnp.exp(z)
    ssum = lax.dot_general(e, jnp.ones((3, 3), jnp.float32),
                           (((1,), (0,)), ((), ())),
                           preferred_element_type=jnp.float32)
    out_ref[...] = e / ssum


def _tc(body, out_shapes, *args):
    return pl.pallas_call(body, out_shape=out_shapes)(*args)


NBLK = 8
BR = NPAD // NBLK


def _bs(shape, blocked_dim=None):
    if blocked_dim is None:
        return pl.BlockSpec(shape, lambda i: tuple(0 for _ in shape))
    block = tuple(BR if d == blocked_dim else n for d, n in enumerate(shape))
    idx = lambda i, _d=blocked_dim: tuple(i if d == _d else 0
                                          for d in range(len(shape)))
    return pl.BlockSpec(block, idx)


def _tcg(body, out_shapes, out_specs, in_specs, *args):
    return pl.pallas_call(body, grid=(NBLK,), out_shape=out_shapes,
                          out_specs=out_specs, in_specs=in_specs)(*args)


_TABS = jax.ShapeDtypeStruct((NQ, NPAD, QW), jnp.float32)
_TAB_SPEC = _bs((NQ, NPAD, QW), blocked_dim=1)
_AGG_SPEC = _bs((NQ, NC, NPAD, QW), blocked_dim=2)
_DINV_SPEC = _bs((NPAD, H), blocked_dim=0)


def kernel(x, edge_index, W1, b1, gW, gb, W2, b2, W3, b3,
           cW1, cb1, cW2, cb2, cW3, cb3):
    n = x.shape[0]
    e = edge_index.shape[1]
    epad = (-e) % (NW * CB * CH)
    kchunks = (e + epad) // (NW * CH)
    kb = kchunks // CB

    pad = jnp.full((epad,), n, dtype=edge_index.dtype)
    srcb = jnp.concatenate([edge_index[0], pad]).reshape(NW, kb, CB, CH)
    dstb = jnp.concatenate([edge_index[1], pad]).reshape(NW, kb, CB, CH)
    xp = jnp.pad(x, ((0, NPAD - n), (0, 0)))

    ones_d = jnp.ones((CH, DW), jnp.float32)
    zeros_d = jnp.zeros((RPS, DW), jnp.float32)
    zeros_q = jnp.zeros((RPS, QW), jnp.float32)

    b1r = b1.reshape(1, -1)
    gwr = jnp.tile(gW, (H, 1))
    gbr = jnp.tile(gb.reshape(1, 1), (1, H))
    b2r = b2.reshape(1, -1)
    b3r = b3.reshape(1, -1)
    cb1r = cb1.reshape(1, -1)
    cb2r = cb2.reshape(1, -1)
    cb3r = cb3.reshape(1, -1)

    degp = _deg_sc(dstb, ones_d, zeros_d, kchunks)

    tab1, dinv = _tcg(
        _prep_body,
        (_TABS, jax.ShapeDtypeStruct((NPAD, H), jnp.float32)),
        (_TAB_SPEC, _DINV_SPEC),
        [_bs((NPAD, 128), 0), _bs((H, 128)), _bs((NC, NPAD, DW), 1)],
        xp, W1, degp)

    agg1 = _agg_sc(tab1, srcb, dstb, zeros_q, kchunks)
    tab2 = _tcg(_mid1_body, _TABS, _TAB_SPEC,
                [_AGG_SPEC, _TAB_SPEC, _DINV_SPEC, _bs((1, H)), _bs((H, H)),
                 _bs((1, H)), _bs((H, H))],
                agg1, tab1, dinv, b1r, gwr, gbr, W2)

    agg2 = _agg_sc(tab2, srcb, dstb, zeros_q, kchunks)
    tab3 = _tcg(_mid2_body, _TABS, _TAB_SPEC,
                [_AGG_SPEC, _TAB_SPEC, _DINV_SPEC, _bs((1, H)), _bs((H, H))],
                agg2, tab2, dinv, b2r, W3)

    agg3 = _agg_sc(tab3, srcb, dstb, zeros_q, kchunks)
    g = _tcg(_gsum_body, jax.ShapeDtypeStruct((1, H), jnp.float32),
             _bs((1, H)),
             [_AGG_SPEC, _TAB_SPEC, _DINV_SPEC, _bs((1, H))],
             agg3, tab3, dinv, b3r)
    out = _tc(_head_body, jax.ShapeDtypeStruct((1, 3), jnp.float32),
              g, cW1, cb1r, cW2, cb2r, cW3, cb3r)
    return out


# speedup vs baseline: 32.8188x; 2.0597x over previous
import functools

import jax
import jax.numpy as jnp
from jax import lax
from jax.experimental import pallas as pl
from jax.experimental.pallas import tpu as pltpu
from jax.experimental.pallas import tpu_sc as plsc

N_NODES = 10000
H = 32
QW = 32
NQ = H // QW
DW = 8

NC = 2
NS = 16
NW = NC * NS
CH = 128
CB = 8
NPAD = 10112
NSP = 10016
RPS = NPAD // NS
LAST = NS - 1
RLAST = NSP - RPS * LAST


def _mesh():
    return plsc.VectorSubcoreMesh(core_axis_name="c", subcore_axis_name="s",
                                  num_cores=NC, num_subcores=NS)


def _tile_slab(s, do):
    @pl.when(s < LAST)
    def _():
        do(s * RPS, RPS)

    @pl.when(s == LAST)
    def _():
        do(LAST * RPS, RLAST)


def _deg_sc(dstb, ones, zeros, kchunks):
    kb = kchunks // CB

    def body(dstb_hbm, ones_hbm, zeros_hbm, out_hbm,
             dst_v, ones_v, buf_v, acc_sh):
        c = lax.axis_index("c")
        s = lax.axis_index("s")
        wid = c * NS + s
        pltpu.sync_copy(ones_hbm, ones_v)
        pltpu.sync_copy(zeros_hbm, buf_v)

        def zacc(start, size):
            pltpu.sync_copy(buf_v.at[pl.ds(0, size)],
                            acc_sh.at[pl.ds(start, size)])

        _tile_slab(s, zacc)
        plsc.subcore_barrier()

        def block(j, carry):
            pltpu.sync_copy(dstb_hbm.at[wid, j], dst_v)
            for r in range(CB):
                pltpu.sync_copy(ones_v, acc_sh.at[dst_v.at[r]], add=True)
            return carry

        lax.fori_loop(0, kb, block, 0)
        plsc.subcore_barrier()

        def wout(start, size):
            pltpu.sync_copy(acc_sh.at[pl.ds(start, size)],
                            buf_v.at[pl.ds(0, size)])
            pltpu.sync_copy(buf_v.at[pl.ds(0, size)],
                            out_hbm.at[c, pl.ds(start, size)])

        _tile_slab(s, wout)

    fn = pl.kernel(
        body,
        out_type=jax.ShapeDtypeStruct((NC, NPAD, DW), jnp.float32),
        mesh=_mesh(),
        compiler_params=pltpu.CompilerParams(use_tc_tiling_on_sc=False),
        scratch_types=[
            pltpu.VMEM((CB, CH), jnp.int32),
            pltpu.VMEM((CH, DW), jnp.float32),
            pltpu.VMEM((RPS, DW), jnp.float32),
            pltpu.VMEM_SHARED((NSP, DW), jnp.float32),
        ],
    )
    return fn(dstb, ones, zeros)


@functools.lru_cache(maxsize=None)
def _agg_fn(kchunks):
    kb = kchunks // CB

    def body(table_hbm, srcb_hbm, dstb_hbm, zeros_hbm, out_hbm,
             src_v, dst_v, rows_v, buf_v, zero_v, tab_sh, acc_sh):
        c = lax.axis_index("c")
        s = lax.axis_index("s")
        wid = c * NS + s
        pltpu.sync_copy(zeros_hbm, zero_v)

        for p in range(NQ):
            def stage(start, size):
                pltpu.sync_copy(table_hbm.at[p, pl.ds(start, size)],
                                buf_v.at[pl.ds(0, size)])
                pltpu.sync_copy(buf_v.at[pl.ds(0, size)],
                                tab_sh.at[pl.ds(start, size)])

            def zacc(start, size):
                pltpu.sync_copy(zero_v.at[pl.ds(0, size)],
                                acc_sh.at[pl.ds(start, size)])

            _tile_slab(s, stage)
            _tile_slab(s, zacc)
            plsc.subcore_barrier()

            def block(j, carry):
                pltpu.sync_copy(srcb_hbm.at[wid, j], src_v)
                pltpu.sync_copy(dstb_hbm.at[wid, j], dst_v)
                for r in range(CB):
                    pltpu.sync_copy(tab_sh.at[src_v.at[r]], rows_v)
                    pltpu.sync_copy(rows_v, acc_sh.at[dst_v.at[r]],
                                    add=True)
                return carry

            lax.fori_loop(0, kb, block, 0)
            plsc.subcore_barrier()

            def wout(start, size):
                pltpu.sync_copy(acc_sh.at[pl.ds(start, size)],
                                buf_v.at[pl.ds(0, size)])
                pltpu.sync_copy(buf_v.at[pl.ds(0, size)],
                                out_hbm.at[p, c, pl.ds(start, size)])

            _tile_slab(s, wout)
            if p != NQ - 1:
                plsc.subcore_barrier()

    return pl.kernel(
        body,
        out_type=jax.ShapeDtypeStruct((NQ, NC, NPAD, QW), jnp.float32),
        mesh=_mesh(),
        compiler_params=pltpu.CompilerParams(use_tc_tiling_on_sc=False),
        scratch_types=[
            pltpu.VMEM((CB, CH), jnp.int32),
            pltpu.VMEM((CB, CH), jnp.int32),
            pltpu.VMEM((CH, QW), jnp.float32),
            pltpu.VMEM((RPS, QW), jnp.float32),
            pltpu.VMEM((RPS, QW), jnp.float32),
            pltpu.VMEM_SHARED((NSP, QW), jnp.float32),
            pltpu.VMEM_SHARED((NSP, QW), jnp.float32),
        ],
    )


def _agg_sc(tabs, srcb, dstb, zeros, kchunks):
    return _agg_fn(kchunks)(tabs, srcb, dstb, zeros)


def _ct(a, b):
    return lax.dot_general(a, b, (((1,), (1,)), ((), ())),
                           preferred_element_type=jnp.float32)


def _split(hs, out_ref):
    for p in range(NQ):
        out_ref[p] = hs[:, p * QW:(p + 1) * QW]


def _join_agg(aggp):
    return jnp.concatenate([aggp[p, 0] + aggp[p, 1] for p in range(NQ)],
                           axis=1)


def _join_tab(tabs):
    return jnp.concatenate([tabs[p] for p in range(NQ)], axis=1)


def _prep_body(x_ref, w1_ref, degp_ref, tab1_ref, dinv_ref):
    degq = degp_ref[0] + degp_ref[1]
    rep = jnp.full((H, DW), 1.0 / DW, jnp.float32)
    deg = _ct(degq, rep) + 1.0
    dinv = lax.rsqrt(deg)
    h = _ct(x_ref[...], w1_ref[...])
    _split(h * dinv, tab1_ref)
    dinv_ref[...] = dinv


def _mid1_body(aggp_ref, tab1_ref, dinv_ref, b1_ref, gw_ref, gb_ref, w2_ref,
               tab2_ref):
    dinv = dinv_ref[...]
    h1 = jax.nn.relu((_join_agg(aggp_ref[...]) + _join_tab(tab1_ref[...]))
                     * dinv + b1_ref[...])
    gate = jax.nn.sigmoid(_ct(h1, gw_ref[...]) + gb_ref[...])
    h1g = h1 * gate
    _split(_ct(h1g, w2_ref[...]) * dinv, tab2_ref)


def _mid2_body(aggp_ref, tab2_ref, dinv_ref, b2_ref, w3_ref, tab3_ref):
    dinv = dinv_ref[...]
    h2 = jax.nn.relu((_join_agg(aggp_ref[...]) + _join_tab(tab2_ref[...]))
                     * dinv + b2_ref[...])
    _split(_ct(h2, w3_ref[...]) * dinv, tab3_ref)


def _gsum_body(aggp_ref, tab3_ref, dinv_ref, b3_ref, gsum_ref):
    i = pl.program_id(0)
    h3 = ((_join_agg(aggp_ref[...]) + _join_tab(tab3_ref[...]))
          * dinv_ref[...] + b3_ref[...])
    rows = lax.broadcasted_iota(jnp.int32, (BR, H), 0) + i * BR
    h3 = jnp.where(rows < N_NODES, h3, 0.0)
    part = jnp.sum(h3, axis=0, keepdims=True) * (1.0 / N_NODES)

    @pl.when(i == 0)
    def _():
        gsum_ref[...] = part

    @pl.when(i > 0)
    def _():
        gsum_ref[...] = gsum_ref[...] + part


def _head_body(g_ref, cw1_ref, cb1_ref, cw2_ref, cb2_ref, cw3_ref, cb3_ref,
               out_ref):
    z = jax.nn.relu(_ct(g_ref[...], cw1_ref[...]) + cb1_ref[...])
    z = jax.nn.relu(_ct(z, cw2_ref[...]) + cb2_ref[...])
    z = _ct(z, cw3_ref[...]) + cb3_ref[...]
    e = jnp.exp(z)
    ssum = lax.dot_general(e, jnp.ones((3, 3), jnp.float32),
                           (((1,), (0,)), ((), ())),
                           preferred_element_type=jnp.float32)
    out_ref[...] = e / ssum


def _tc(body, out_shapes, *args):
    return pl.pallas_call(body, out_shape=out_shapes)(*args)


NBLK = 8
BR = NPAD // NBLK


def _bs(shape, blocked_dim=None):
    if blocked_dim is None:
        return pl.BlockSpec(shape, lambda i: tuple(0 for _ in shape))
    block = tuple(BR if d == blocked_dim else n for d, n in enumerate(shape))
    idx = lambda i, _d=blocked_dim: tuple(i if d == _d else 0
                                          for d in range(len(shape)))
    return pl.BlockSpec(block, idx)


def _tcg(body, out_shapes, out_specs, in_specs, *args):
    return pl.pallas_call(body, grid=(NBLK,), out_shape=out_shapes,
                          out_specs=out_specs, in_specs=in_specs)(*args)


_TABS = jax.ShapeDtypeStruct((NQ, NPAD, QW), jnp.float32)
_TAB_SPEC = _bs((NQ, NPAD, QW), blocked_dim=1)
_AGG_SPEC = _bs((NQ, NC, NPAD, QW), blocked_dim=2)
_DINV_SPEC = _bs((NPAD, H), blocked_dim=0)


def kernel(x, edge_index, W1, b1, gW, gb, W2, b2, W3, b3,
           cW1, cb1, cW2, cb2, cW3, cb3):
    n = x.shape[0]
    e = edge_index.shape[1]
    epad = (-e) % (NW * CB * CH)
    kchunks = (e + epad) // (NW * CH)
    kb = kchunks // CB

    pad = jnp.full((epad,), n, dtype=edge_index.dtype)
    srcb = jnp.concatenate([edge_index[0], pad]).reshape(NW, kb, CB, CH)
    dstb = jnp.concatenate([edge_index[1], pad]).reshape(NW, kb, CB, CH)
    xp = jnp.pad(x, ((0, NPAD - n), (0, 0)))

    ones_d = jnp.ones((CH, DW), jnp.float32)
    zeros_d = jnp.zeros((RPS, DW), jnp.float32)
    zeros_q = jnp.zeros((RPS, QW), jnp.float32)

    b1r = b1.reshape(1, -1)
    gwr = jnp.tile(gW, (H, 1))
    gbr = jnp.tile(gb.reshape(1, 1), (1, H))
    b2r = b2.reshape(1, -1)
    b3r = b3.reshape(1, -1)
    cb1r = cb1.reshape(1, -1)
    cb2r = cb2.reshape(1, -1)
    cb3r = cb3.reshape(1, -1)

    degp = _deg_sc(dstb, ones_d, zeros_d, kchunks)

    tab1, dinv = _tcg(
        _prep_body,
        (_TABS, jax.ShapeDtypeStruct((NPAD, H), jnp.float32)),
        (_TAB_SPEC, _DINV_SPEC),
        [_bs((NPAD, 128), 0), _bs((H, 128)), _bs((NC, NPAD, DW), 1)],
        xp, W1, degp)

    agg1 = _agg_sc(tab1, srcb, dstb, zeros_q, kchunks)
    tab2 = _tcg(_mid1_body, _TABS, _TAB_SPEC,
                [_AGG_SPEC, _TAB_SPEC, _DINV_SPEC, _bs((1, H)), _bs((H, H)),
                 _bs((1, H)), _bs((H, H))],
                agg1, tab1, dinv, b1r, gwr, gbr, W2)

    agg2 = _agg_sc(tab2, srcb, dstb, zeros_q, kchunks)
    tab3 = _tcg(_mid2_body, _TABS, _TAB_SPEC,
                [_AGG_SPEC, _TAB_SPEC, _DINV_SPEC, _bs((1, H)), _bs((H, H))],
                agg2, tab2, dinv, b2r, W3)

    agg3 = _agg_sc(tab3, srcb, dstb, zeros_q, kchunks)
    g = _tcg(_gsum_body, jax.ShapeDtypeStruct((1, H), jnp.float32),
             _bs((1, H)),
             [_AGG_SPEC, _TAB_SPEC, _DINV_SPEC, _bs((1, H))],
             agg3, tab3, dinv, b3r)
    out = _tc(_head_body, jax.ShapeDtypeStruct((1, 3), jnp.float32),
              g, cW1, cb1r, cW2, cb2r, cW3, cb3r)
    return out
